# Initial kernel scaffold; baseline (speedup 1.0000x reference)
import functools

import jax
import jax.numpy as jnp
from jax import lax
from jax.experimental import pallas as pl
from jax.experimental.pallas import tpu as pltpu
from jax.experimental.pallas import tpu_sc as plsc

_D = 64
_B = 16384
_L = 50
_TOTAL = _B * _L
_NC = 2
_NS = 16
_NW = _NC * _NS
_PER_W = _TOTAL // _NW
_G = 128
_NG = _PER_W // _G

_mesh = plsc.VectorSubcoreMesh(core_axis_name="c", subcore_axis_name="s")


@functools.partial(
    pl.kernel,
    out_type=jax.ShapeDtypeStruct((_TOTAL, _D), jnp.float32),
    scratch_types=[
        pltpu.VMEM((_NG, _G), jnp.int32),
        pltpu.VMEM((_G, _D), jnp.float32),
        pltpu.SemaphoreType.DMA,
    ],
    mesh=_mesh,
)
def _emb_lookup(x_hbm, table_hbm, out_hbm, idx_v, rows_v, sem):
    wid = lax.axis_index("s") * _NC + lax.axis_index("c")
    pltpu.sync_copy(x_hbm.at[wid], idx_v)
    base = wid * _PER_W

    def step(g, carry):
        pltpu.async_copy(table_hbm.at[idx_v.at[g]], rows_v, sem).wait()
        pltpu.sync_copy(rows_v, out_hbm.at[pl.ds(base + g * _G, _G)])
        return carry

    lax.fori_loop(0, _NG, step, 0)


def kernel(x, table):
    xf = x.astype(jnp.int32).reshape(_NW, _NG, _G)
    out = _emb_lookup(xf, table)
    return out.reshape(_B, _L, _D)


# speedup vs baseline: 1.6847x; 1.6847x over previous
import functools

import jax
import jax.numpy as jnp
from jax import lax
from jax.experimental import pallas as pl
from jax.experimental.pallas import tpu as pltpu
from jax.experimental.pallas import tpu_sc as plsc

_D = 64
_B = 16384
_L = 50
_TOTAL = _B * _L
_NC = 2
_NS = 16
_NW = _NC * _NS
_PER_W = _TOTAL // _NW
_G = 128
_NG = _PER_W // _G

_mesh = plsc.VectorSubcoreMesh(core_axis_name="c", subcore_axis_name="s")


@functools.partial(
    pl.kernel,
    out_type=jax.ShapeDtypeStruct((_TOTAL, _D), jnp.float32),
    scratch_types=[
        pltpu.VMEM((_NG, _G), jnp.int32),
        pltpu.VMEM((_G, _D), jnp.float32),
        pltpu.SemaphoreType.DMA,
    ],
    mesh=_mesh,
    compiler_params=pltpu.CompilerParams(use_tc_tiling_on_sc=False),
)
def _emb_lookup(x_hbm, table_hbm, out_hbm, idx_v, rows_v, sem):
    wid = lax.axis_index("s") * _NC + lax.axis_index("c")
    pltpu.sync_copy(x_hbm.at[wid], idx_v)
    base = wid * _PER_W

    def step(g, carry):
        pltpu.async_copy(table_hbm.at[idx_v.at[g]], rows_v, sem).wait()
        pltpu.sync_copy(rows_v, out_hbm.at[pl.ds(base + g * _G, _G)])
        return carry

    lax.fori_loop(0, _NG, step, 0)


def kernel(x, table):
    xf = x.astype(jnp.int32).reshape(_NW, _NG, _G)
    out = _emb_lookup(xf, table)
    return out.reshape(_B, _L, _D)


# speedup vs baseline: 1.8775x; 1.1144x over previous
import functools

import jax
import jax.numpy as jnp
from jax import lax
from jax.experimental import pallas as pl
from jax.experimental.pallas import tpu as pltpu
from jax.experimental.pallas import tpu_sc as plsc

_D = 64
_B = 16384
_L = 50
_TOTAL = _B * _L
_NC = 2
_NS = 16
_NW = _NC * _NS
_PER_W = _TOTAL // _NW
_G = 128
_NG = _PER_W // _G

_mesh = plsc.VectorSubcoreMesh(core_axis_name="c", subcore_axis_name="s")


_NBUF = 8
_J = 6
_K = _NBUF - _J


@functools.partial(
    pl.kernel,
    out_type=jax.ShapeDtypeStruct((_TOTAL, _D), jnp.float32),
    scratch_types=[
        pltpu.VMEM((_NG, _G), jnp.int32),
        pltpu.VMEM((_NBUF, _G, _D), jnp.float32),
        pltpu.SemaphoreType.DMA((_NBUF,)),
        pltpu.SemaphoreType.DMA((_NBUF,)),
    ],
    mesh=_mesh,
    compiler_params=pltpu.CompilerParams(use_tc_tiling_on_sc=False),
)
def _emb_lookup(x_hbm, table_hbm, out_hbm, idx_v, rows_v, gsem, ssem):
    wid = lax.axis_index("s") * _NC + lax.axis_index("c")
    pltpu.sync_copy(x_hbm.at[wid], idx_v)
    base = wid * _PER_W

    def gather_desc(g, b):
        return pltpu.make_async_copy(
            table_hbm.at[idx_v.at[g]], rows_v.at[b], gsem.at[b])

    def store_desc(g, b):
        return pltpu.make_async_copy(
            rows_v.at[b], out_hbm.at[pl.ds(base + g * _G, _G)], ssem.at[b])

    for j in range(_J):
        gather_desc(j, j).start()

    def outer(i, carry):
        for j in range(_NBUF):
            g = i * _NBUF + j
            bs = (j - _K) % _NBUF

            @pl.when(g >= _K)
            def _():
                store_desc(g - _K, bs).wait()

            bg = (j + _J) % _NBUF

            @pl.when(g + _J < _NG)
            def _():
                gather_desc(g + _J, bg).start()

            gather_desc(g, j).wait()
            store_desc(g, j).start()
        return carry

    lax.fori_loop(0, _NG // _NBUF, outer, 0)

    for j in range(_K):
        g = _NG - _K + j
        store_desc(g, g % _NBUF).wait()


def kernel(x, table):
    xf = x.astype(jnp.int32).reshape(_NW, _NG, _G)
    out = _emb_lookup(xf, table)
    return out.reshape(_B, _L, _D)
